# Initial kernel scaffold; baseline (speedup 1.0000x reference)
import jax
import jax.numpy as jnp
from jax.experimental import pallas as pl


def kernel(x, u, mask, edge_index, params):
    raise NotImplementedError("write your pallas kernel here")


# speedup vs baseline: 18.1723x; 18.1723x over previous
import functools

import jax
import jax.numpy as jnp
from jax import lax
from jax.experimental import pallas as pl
from jax.experimental.pallas import tpu as pltpu
from jax.experimental.pallas import tpu_sc as plsc

N = 100000
B = 2
BN = B * N
E = 1600000
EB = B * E
EBP = EB + 128
HID = 32
NLAYERS = 4
ETA = 3

NC = 2
NS = 16
NLOC = 6256
DUMP = NLOC
HALF = N
SWEEP = 50000
SPROWS = 50048
DUMP3 = 50040
KE = 64

_I16 = lambda: lax.iota(jnp.int32, 16)


def _shift(v, idx):
    return v.at[idx].get(mode="promise_in_bounds")


def _lanesum(v):
    it = _I16()
    for k in (1, 2, 4, 8):
        idx = (it + k) & 15
        v = v + _shift(v, idx)
    return v


def _splat(v, lane):
    return _shift(v, jnp.full((16,), lane, jnp.int32))


def _seg_scan(d, m, prev_d, prev_m, is_max):
    it = _I16()
    for k in (1, 2, 4, 8):
        idx = jnp.maximum(it - k, 0)
        d_sh = _shift(d, idx)
        m_sh = _shift(m, idx)
        ok = (d_sh == d) & (it >= k)
        if is_max:
            m = jnp.where(ok, jnp.maximum(m, m_sh), m)
        else:
            m = jnp.where(ok, m + m_sh, m)
    okc = d == prev_d
    if is_max:
        m = jnp.where(okc, jnp.maximum(m, prev_m), m)
    else:
        m = jnp.where(okc, m + prev_m, m)
    return m


def _runend_store(loc_ref, d, dl, m, prev_d, prev_dl, prev_m):
    it = _I16()
    idx2 = jnp.minimum(it + 1, 15)
    d_next = _shift(d, idx2)
    isend = (d != d_next) & (it < 15)
    plsc.store_scatter(loc_ref, [dl], m, mask=isend)
    flush = (it == 0) & (d != prev_d) & (prev_d >= 0)
    plsc.store_scatter(loc_ref, [prev_dl], prev_m, mask=flush)
    i15 = jnp.full((16,), 15, jnp.int32)
    return _shift(d, i15), _shift(dl, i15), _shift(m, i15)


def _edge_kernel_body(asrc, adst, obsf, fs, fd, wb, att,
                      acc_out, den_out, smax_out, scores_out,
                      wbv, idx_s, idx_d, rows_s, rows_d, obs_v, sco_v, spi_v,
                      gsm_v, gde_v, akv, attv, smax_loc, den_loc, zbuf,
                      acc_sp, sem, sem2):
    c = lax.axis_index("c")
    s = lax.axis_index("s")
    w = c * NS + s
    it = _I16()

    pltpu.sync_copy(wb, wbv)
    pltpu.sync_copy(att, attv)
    ev = wbv[pl.ds(w, 16)]
    e0 = ev[0]
    e1 = ev[1]

    lo = pl.multiple_of(c * HALF + s * NLOC, 16)
    hi = jnp.minimum(lo + NLOC, (c + 1) * HALF)

    def _zb(i, _):
        den_loc[pl.ds(i * 16, 16)] = jnp.zeros((16,), jnp.float32)
        return 0
    lax.fori_loop(0, (NLOC + 16) // 16, _zb, 0)

    att0 = attv[pl.ds(0, 16)]
    att1 = attv[pl.ds(16, 16)]

    estart = e0 - lax.rem(e0, 16)
    nb = (e1 - estart + KE - 1) // KE

    def _fetch_rows(eb, want_obs):
        eb = pl.multiple_of(eb, 16)
        pltpu.sync_copy(fs.at[pl.ds(eb, KE)], idx_s)
        pltpu.sync_copy(fd.at[pl.ds(eb, KE)], idx_d)
        cp1 = pltpu.async_copy(asrc.at[idx_s], rows_s, sem)
        cp2 = pltpu.async_copy(adst.at[idx_d], rows_d, sem2)
        if want_obs:
            cp3 = pltpu.async_copy(obsf.at[idx_s], obs_v, sem)
        cp1.wait(); cp2.wait()
        if want_obs:
            cp3.wait()

    def _kv2(e):
        z0 = rows_s[e, pl.ds(0, 16)] + rows_d[e, pl.ds(0, 16)]
        z1 = rows_s[e, pl.ds(16, 16)] + rows_d[e, pl.ds(16, 16)]
        return (jnp.maximum(z0, 0.01 * z0), jnp.maximum(z1, 0.01 * z1))

    def _p1(i, carry):
        prev_d, prev_dl, prev_m = carry
        eb = estart + i * KE
        _fetch_rows(eb, True)
        for j in range(KE // 16):
            base = j * 16
            obv = obs_v[pl.ds(base, 16)]
            svec = jnp.zeros((16,), jnp.float32)
            for e in range(16):
                k0, k1 = _kv2(base + e)
                sc = _lanesum(k0 * att0 + k1 * att1)
                svec = jnp.where(it == e, sc, svec)
            svec = jnp.where(obv > 0.0, svec, -1e9)
            sco_v[pl.ds(base, 16)] = svec
            d = idx_d[pl.ds(base, 16)]
            own = (d >= lo) & (d < hi)
            dl = jnp.where(own, d - lo, DUMP)
            m = _seg_scan(d, svec, prev_d, prev_m, True)
            prev_d, prev_dl, prev_m = _runend_store(
                smax_loc, d, dl, m, prev_d, prev_dl, prev_m)
        pltpu.sync_copy(sco_v, scores_out.at[pl.ds(pl.multiple_of(eb, 16), KE)])
        return prev_d, prev_dl, prev_m

    neg = jnp.full((16,), -1, jnp.int32)
    carry0 = (neg, jnp.full((16,), DUMP, jnp.int32),
              jnp.full((16,), -3.0e38, jnp.float32))
    prev_d, prev_dl, prev_m = lax.fori_loop(0, nb, _p1, carry0)
    flush = (it == 0) & (prev_d >= 0)
    plsc.store_scatter(smax_loc, [prev_dl], prev_m, mask=flush)

    def _p2(i, carry):
        prev_d, prev_dl, prev_m = carry
        eb = pl.multiple_of(estart + i * KE, 16)
        pltpu.sync_copy(fd.at[pl.ds(eb, KE)], idx_d)
        pltpu.sync_copy(scores_out.at[pl.ds(eb, KE)], sco_v)
        for j in range(KE // 16):
            d = idx_d[pl.ds(j * 16, 16)]
            own = (d >= lo) & (d < hi)
            dl = jnp.where(own, d - lo, DUMP)
            gsm = plsc.load_gather(smax_loc, [dl])
            ex = jnp.exp(sco_v[pl.ds(j * 16, 16)] - gsm)
            m = _seg_scan(d, ex, prev_d, prev_m, False)
            prev_d, prev_dl, prev_m = _runend_store(
                den_loc, d, dl, m, prev_d, prev_dl, prev_m)
        return prev_d, prev_dl, prev_m

    prev_d, prev_dl, prev_m = lax.fori_loop(0, nb, _p2, carry0)
    flush = (it == 0) & (prev_d >= 0)
    plsc.store_scatter(den_loc, [prev_dl], prev_m, mask=flush)

    @pl.when(s < NS - 1)
    def _():
        pltpu.sync_copy(smax_loc.at[pl.ds(0, NLOC)], smax_out.at[pl.ds(lo, NLOC)])
        pltpu.sync_copy(den_loc.at[pl.ds(0, NLOC)], den_out.at[pl.ds(lo, NLOC)])

    @pl.when(s == NS - 1)
    def _():
        SHORT = HALF - (NS - 1) * NLOC
        pltpu.sync_copy(smax_loc.at[pl.ds(0, SHORT)], smax_out.at[pl.ds(lo, SHORT)])
        pltpu.sync_copy(den_loc.at[pl.ds(0, SHORT)], den_out.at[pl.ds(lo, SHORT)])

    def _zz(i, _):
        zbuf[i, pl.ds(0, 16)] = jnp.zeros((16,), jnp.float32)
        zbuf[i, pl.ds(16, 16)] = jnp.zeros((16,), jnp.float32)
        return 0
    lax.fori_loop(0, zbuf.shape[0], _zz, 0)

    plsc.subcore_barrier()

    for sw in range(2):
        for zi in range(3128 // 92):
            pltpu.sync_copy(zbuf, acc_sp.at[pl.ds(s * 3128 + zi * 92, 92)])
        plsc.subcore_barrier()

        sv = wbv[pl.ds(33 + 2 * c + sw, 16)]
        slo = sv[0]
        shi = sv[1]
        cnt = (shi - slo + NS - 1) // NS
        we0 = slo + s * cnt
        we1 = jnp.minimum(we0 + cnt, shi)
        est3 = we0 - lax.rem(we0, 16)
        nb3 = (we1 - est3 + KE - 1) // KE
        nbase = c * HALF + sw * SWEEP

        def _p3(i, _):
            eb = pl.multiple_of(est3 + i * KE, 16)
            _fetch_rows(eb, False)
            pltpu.sync_copy(scores_out.at[pl.ds(eb, KE)], sco_v)
            cg1 = pltpu.async_copy(smax_out.at[idx_d], gsm_v, sem)
            cg2 = pltpu.async_copy(den_out.at[idx_d], gde_v, sem2)
            cg1.wait(); cg2.wait()
            for j in range(KE // 16):
                base = j * 16
                d = idx_d[pl.ds(base, 16)]
                rid = it + base
                valid = (rid + eb >= we0) & (rid + eb < we1)
                gsm = gsm_v[pl.ds(base, 16)]
                gde = gde_v[pl.ds(base, 16)]
                svec = sco_v[pl.ds(base, 16)]
                alpha = jnp.exp(svec - gsm) / (gde + 1e-16)
                alpha = jnp.where(valid, alpha, 0.0)
                spi = jnp.where(valid, d - nbase, DUMP3)
                spi_v[pl.ds(base, 16)] = spi
                for e in range(16):
                    k0, k1 = _kv2(base + e)
                    a = _splat(alpha, e)
                    akv[base + e, pl.ds(0, 16)] = a * k0
                    akv[base + e, pl.ds(16, 16)] = a * k1
            pltpu.sync_copy(akv, acc_sp.at[spi_v], add=True)
            return 0

        lax.fori_loop(0, nb3, _p3, 0)
        plsc.subcore_barrier()
        pltpu.sync_copy(acc_sp.at[pl.ds(s * 3128, 3128)],
                        acc_out.at[2 * c + sw, pl.ds(s * 3128, 3128)])
        plsc.subcore_barrier()


def _make_edge_kernel():
    mesh = plsc.VectorSubcoreMesh(core_axis_name="c", subcore_axis_name="s")
    return functools.partial(
        pl.kernel,
        mesh=mesh,
        compiler_params=pltpu.CompilerParams(
            needs_layout_passes=False, use_tc_tiling_on_sc=False),
        out_type=[
            jax.ShapeDtypeStruct((4, SPROWS, HID), jnp.float32),
            jax.ShapeDtypeStruct((BN,), jnp.float32),
            jax.ShapeDtypeStruct((BN,), jnp.float32),
            jax.ShapeDtypeStruct((EBP,), jnp.float32),
        ],
        scratch_types=[
            pltpu.VMEM((64,), jnp.int32),
            pltpu.VMEM((KE,), jnp.int32),
            pltpu.VMEM((KE,), jnp.int32),
            pltpu.VMEM((KE, HID), jnp.float32),
            pltpu.VMEM((KE, HID), jnp.float32),
            pltpu.VMEM((KE,), jnp.float32),
            pltpu.VMEM((KE,), jnp.float32),
            pltpu.VMEM((KE,), jnp.int32),
            pltpu.VMEM((KE,), jnp.float32),
            pltpu.VMEM((KE,), jnp.float32),
            pltpu.VMEM((KE, HID), jnp.float32),
            pltpu.VMEM((HID,), jnp.float32),
            pltpu.VMEM((NLOC + 16,), jnp.float32),
            pltpu.VMEM((NLOC + 16,), jnp.float32),
            pltpu.VMEM((92, HID), jnp.float32),
            pltpu.VMEM_SHARED((SPROWS, HID), jnp.float32),
            pltpu.SemaphoreType.DMA,
            pltpu.SemaphoreType.DMA,
        ],
    )(_edge_kernel_body)


RB = 1000
GRID = BN // RB


def _ln(x, g, b):
    mu = jnp.mean(x, axis=-1, keepdims=True)
    var = jnp.mean((x - mu) * (x - mu), axis=-1, keepdims=True)
    return (x - mu) / jnp.sqrt(var + 1e-5) * g + b


def _rowspec(width):
    return pl.BlockSpec((RB, width), lambda i: (i, 0))


def _embspec(width):
    nb = N // RB
    return pl.BlockSpec((RB, width), lambda i: (i % nb, 0))


def _wspec(shape):
    if len(shape) == 1:
        return pl.BlockSpec((1, shape[0]), lambda i: (0, 0))
    return pl.BlockSpec(shape, lambda i: (0, 0))


def _stage_a_body(x, u, mask, node_emb, Wu1, bu1, Wu2, bu2, Wh1, bh1,
                  Wh2, bh2, gn, bn, Wsk, bsk, Wsrc, Wdst, batt,
                  h_out, asrc_out, adst_out):
    maskf = mask[...]
    xm = x[...] * maskf
    q = jnp.maximum(u[...] @ Wu1[...] + bu1[...], 0.0) @ Wu2[...] + bu2[...] \
        + node_emb[...]
    h = jnp.maximum(xm @ Wh1[...] + bh1[...], 0.0) @ Wh2[...] + bh2[...] + q
    h = jnp.where(maskf > 0.0, h, q)
    h = _ln(h, gn[...], bn[...])
    hl = h + (xm @ Wsk[...] + bsk[...]) * maskf
    h_out[...] = hl
    asrc_out[...] = hl @ Wsrc[...]
    adst_out[...] = hl @ Wdst[...] + batt[...]


def _stage_c_body(with_next, next_eta, h, acc, den, x, mask, vemb, memb,
                  Wmsg, bmsg, Wroot, broot, g, b, Wr1, br1, Wr2, br2,
                  Wsk, bsk, Wsrc, Wdst, batt,
                  ro_out, h_out=None, asrc_out=None, adst_out=None):
    hv = h[...]
    denv = den[...]
    sa = denv / (denv + 1e-16)
    agg = acc[0] @ Wmsg[...] + sa * bmsg[...]
    out = hv @ Wroot[...] + broot[...] + agg
    out = _ln(out, g[...], b[...])
    ro = jnp.maximum(out @ Wr1[...] + br1[...], 0.0) @ Wr2[...] + br2[...]
    ro_out[...] = ro
    if with_next:
        maskf = mask[...]
        xm = x[...] * maskf
        if next_eta:
            out = out + jnp.where(maskf > 0.0, vemb[...], memb[...])
        hl = out + (xm @ Wsk[...] + bsk[...]) * maskf
        h_out[...] = hl
        asrc_out[...] = hl @ Wsrc[...]
        adst_out[...] = hl @ Wdst[...] + batt[...]


def _accspec():
    nchunk = SWEEP // RB
    return pl.BlockSpec((1, RB, HID), lambda i: (i // nchunk, i % nchunk, 0))


def kernel(x, u, mask, edge_index, params):
    p = params
    f32 = jnp.float32

    order = jnp.argsort(edge_index[1])
    es = edge_index[0][order]
    ed = edge_index[1][order]
    fs = jnp.concatenate([es, es + N, jnp.zeros((128,), jnp.int32)])
    fd = jnp.concatenate([ed, ed + N, jnp.full((128,), BN, jnp.int32)])
    starts = jnp.array(
        [c * HALF + s * NLOC for c in range(NC) for s in range(NS)] + [BN],
        jnp.int32)
    wb33 = jnp.searchsorted(fd, starts, side="left").astype(jnp.int32)
    sw5 = jnp.searchsorted(
        fd, jnp.array([0, SWEEP, 2 * SWEEP, 3 * SWEEP, 4 * SWEEP], jnp.int32),
        side="left").astype(jnp.int32)
    wb = jnp.concatenate([wb33, sw5, jnp.zeros((26,), jnp.int32)])

    xf = x.reshape(BN, 1)
    uf = u.reshape(BN, 16)
    maskf = mask.astype(f32).reshape(BN, 1)

    edge_call = _make_edge_kernel()

    a_out = pl.pallas_call(
        _stage_a_body,
        grid=(GRID,),
        in_specs=[
            _rowspec(1), _rowspec(16), _rowspec(1), _embspec(HID),
            _wspec((16, HID)), _wspec((HID,)), _wspec((HID, HID)), _wspec((HID,)),
            _wspec((1, HID)), _wspec((HID,)), _wspec((HID, HID)), _wspec((HID,)),
            _wspec((HID,)), _wspec((HID,)),
            _wspec((1, HID)), _wspec((HID,)),
            _wspec((HID, HID)), _wspec((HID, HID)), _wspec((HID,)),
        ],
        out_specs=[_rowspec(HID), _rowspec(HID), _rowspec(HID)],
        out_shape=[jax.ShapeDtypeStruct((BN, HID), f32)] * 3,
    )
    lp0 = p['layers'][0]
    h, asrc, adst = a_out(
        xf, uf, maskf, p['node_emb'],
        p['Wu1'], p['bu1'].reshape(1, -1), p['Wu2'], p['bu2'].reshape(1, -1),
        p['Wh1'], p['bh1'].reshape(1, -1), p['Wh2'], p['bh2'].reshape(1, -1),
        p['gn'].reshape(1, -1), p['bn'].reshape(1, -1),
        lp0['Wsk'], lp0['bsk'].reshape(1, -1),
        lp0['Wsrc'], lp0['Wdst'], lp0['batt'].reshape(1, -1))

    obsf = maskf.reshape(BN)
    ones = jnp.ones((BN,), f32)
    ros = []
    for L in range(NLAYERS):
        lp = p['layers'][L]
        obs_l = obsf if L < ETA else ones
        acc, den, smax, scores = edge_call(
            asrc, adst, obs_l, fs, fd, wb, lp['att'])

        with_next = L < NLAYERS - 1
        next_eta = (L + 1) == ETA
        lpn = p['layers'][L + 1] if with_next else lp
        body = functools.partial(_stage_c_body, with_next, next_eta)
        out_specs = [_rowspec(1)]
        out_shape = [jax.ShapeDtypeStruct((BN, 1), f32)]
        if with_next:
            out_specs += [_rowspec(HID)] * 3
            out_shape += [jax.ShapeDtypeStruct((BN, HID), f32)] * 3
        c_out = pl.pallas_call(
            body,
            grid=(GRID,),
            in_specs=[
                _rowspec(HID), _accspec(), _rowspec(1), _rowspec(1), _rowspec(1),
                _embspec(HID), _embspec(HID),
                _wspec((HID, HID)), _wspec((HID,)),
                _wspec((HID, HID)), _wspec((HID,)),
                _wspec((HID,)), _wspec((HID,)),
                _wspec((HID, HID)), _wspec((HID,)),
                _wspec((HID, 1)), _wspec((1,)),
                _wspec((1, HID)), _wspec((HID,)),
                _wspec((HID, HID)), _wspec((HID, HID)), _wspec((HID,)),
            ],
            out_specs=out_specs,
            out_shape=out_shape,
        )
        res = c_out(
            h, acc, den.reshape(BN, 1), xf, maskf,
            p['valid_emb'], p['mask_emb'],
            lp['Wmsg'], lp['bmsg'].reshape(1, -1),
            lp['Wroot'], lp['broot'].reshape(1, -1),
            lp['g'].reshape(1, -1), lp['b'].reshape(1, -1),
            lp['Wr1'], lp['br1'].reshape(1, -1),
            lp['Wr2'], lp['br2'].reshape(1, -1),
            lpn['Wsk'], lpn['bsk'].reshape(1, -1),
            lpn['Wsrc'], lpn['Wdst'], lpn['batt'].reshape(1, -1))
        if with_next:
            ro, h, asrc, adst = res
        else:
            ro = res[0]
        ros.append(ro.reshape(B, N, 1))

    x_hat = ros.pop(-1)
    return (x_hat, ros[0], ros[1], ros[2])
